# baseline (speedup 1.0000x reference)
import math

import jax
import jax.numpy as jnp
from jax.experimental import pallas as pl
from jax.experimental.pallas import tpu as pltpu

_INT_MIN = -(2 ** 31)
_INT_MAX = 2 ** 31 - 1


def _sortable(f):
    b = jax.lax.bitcast_convert_type(f, jnp.int32)
    return jnp.where(b < 0,
                     jnp.bitwise_xor(jnp.bitwise_not(b), jnp.int32(_INT_MIN)),
                     b)


def _gelu(x):
    return x * jax.nn.sigmoid(
        x * (1.5957691216057308 + 0.07135481627272282 * (x * x)))


def _search_all(keys, kk, hw):
    nb, r, cc = keys.shape

    def cnt_gt(thrv):
        return jnp.sum((keys > thrv).astype(jnp.int32), axis=(1, 2),
                       keepdims=True)

    cnt_nonneg = jnp.sum((keys >= 0).astype(jnp.int32), axis=(1, 2),
                         keepdims=True)
    lo0 = jnp.where(cnt_nonneg >= kk,
                    jnp.zeros_like(cnt_nonneg),
                    jnp.full_like(cnt_nonneg, _INT_MIN))
    hi0 = jnp.where(cnt_nonneg >= kk,
                    jnp.full_like(cnt_nonneg, _INT_MAX),
                    jnp.full_like(cnt_nonneg, -1))

    def bs(i, lh):
        lo, hi = lh
        mid = lo + ((hi - lo) >> 1)
        c = cnt_gt(mid)
        take = c < kk
        return (jnp.where(take, lo, mid + 1), jnp.where(take, mid, hi))

    thr, _ = jax.lax.fori_loop(0, 31, bs, (lo0, hi0))
    rem = kk - cnt_gt(thr)
    eq = keys == thr

    ids = (jax.lax.broadcasted_iota(jnp.int32, (nb, r, cc), 1) * cc
           + jax.lax.broadcasted_iota(jnp.int32, (nb, r, cc), 2))

    def bs2(i, lh):
        lo2, hi2 = lh
        mid = (lo2 + hi2) >> 1
        c = jnp.sum((eq & (ids <= mid)).astype(jnp.int32), axis=(1, 2),
                    keepdims=True)
        take = c >= rem
        return (jnp.where(take, lo2, mid + 1), jnp.where(take, mid, hi2))

    z = jnp.zeros_like(thr)
    m, _ = jax.lax.fori_loop(0, 16, bs2, (z, z + (hw - 1)))
    return thr, jnp.where(rem > 0, m, z - 1)


def _make_kernel(nb, c, hw, tile, kk, nbuf):
    nt = hw // tile
    steps = nb * nt

    def body(prop8_ref, x_ref, p_ref, w1t_ref, b1_ref, w2t_ref, b2_ref,
             onesc_ref, out_ref, ibuf, pbuf, obuf, isem, psem, osem, sref):
        for k in range(nbuf):
            b0, t0 = k // nt, k % nt
            pltpu.make_async_copy(
                x_ref.at[b0, :, pl.ds(t0 * tile, tile)],
                ibuf.at[k], isem.at[k]).start()
            pltpu.make_async_copy(
                p_ref.at[b0, :, pl.ds(t0 * tile, tile)],
                pbuf.at[k], psem.at[k]).start()

        thr_all, m_all = _search_all(_sortable(prop8_ref[...]), kk, hw)
        biota = jax.lax.broadcasted_iota(jnp.int32, (nb, 1, 1), 0)
        for b in range(nb):
            sel = biota == b
            sref[b, 0] = jnp.sum(jnp.where(sel, thr_all, 0))
            sref[b, 1] = jnp.sum(jnp.where(sel, m_all, 0))

        def step(j, k):
            s = j * nbuf + k
            b = s // nt
            t = s % nt
            thr = sref[b, 0]
            m = sref[b, 1]

            pltpu.make_async_copy(
                x_ref.at[b, :, pl.ds(t * tile, tile)],
                ibuf.at[k], isem.at[k]).wait()
            pltpu.make_async_copy(
                p_ref.at[b, :, pl.ds(t * tile, tile)],
                pbuf.at[k], psem.at[k]).wait()

            keys_t = _sortable(pbuf[k])
            ids_t = (jax.lax.broadcasted_iota(jnp.int32, (1, tile), 1)
                     + t * tile)
            mask = ((keys_t > thr) | ((keys_t == thr) & (ids_t <= m))
                    ).astype(jnp.float32)

            h = ibuf[k]
            mu = jnp.dot(onesc_ref[...], h, preferred_element_type=jnp.float32)
            ms = jnp.dot(onesc_ref[...], h * h,
                         preferred_element_type=jnp.float32)
            r = jax.lax.rsqrt(ms - mu * mu + 1e-5)
            zn = ((h - mu) * r).astype(jnp.bfloat16)
            z1 = jnp.dot(w1t_ref[...], zn,
                         preferred_element_type=jnp.float32) + b1_ref[...]
            a = _gelu(z1).astype(jnp.bfloat16)
            ff = jnp.dot(w2t_ref[...], a,
                         preferred_element_type=jnp.float32) + b2_ref[...]

            @pl.when(j > 0)
            def _wait_prev_out():
                pltpu.make_async_copy(
                    obuf.at[k], out_ref.at[b, :, pl.ds(t * tile, tile)],
                    osem.at[k]).wait()

            obuf[k] = h + mask * ff
            pltpu.make_async_copy(
                obuf.at[k], out_ref.at[b, :, pl.ds(t * tile, tile)],
                osem.at[k]).start()

            @pl.when(s + nbuf < steps)
            def _fetch_ahead():
                s2 = s + nbuf
                b2 = s2 // nt
                t2 = s2 % nt
                pltpu.make_async_copy(
                    x_ref.at[b2, :, pl.ds(t2 * tile, tile)],
                    ibuf.at[k], isem.at[k]).start()
                pltpu.make_async_copy(
                    p_ref.at[b2, :, pl.ds(t2 * tile, tile)],
                    pbuf.at[k], psem.at[k]).start()

        def loop_body(j, carry):
            for k in range(nbuf):
                step(j, k)
            return carry

        jax.lax.fori_loop(0, steps // nbuf, loop_body, 0)

        for k in range(nbuf):
            s = steps - nbuf + k
            b0 = s // nt
            t0 = s % nt
            pltpu.make_async_copy(
                obuf.at[k], out_ref.at[b0, :, pl.ds(t0 * tile, tile)],
                osem.at[k]).wait()

    return body


def kernel(x, proposal, ln_g0, ln_b0, w1_0, b1_0, w2_0, b2_0):
    B, C, H, W = x.shape
    HW = H * W
    HID = w1_0.shape[1]
    kk = max(int(math.ceil(HW * 0.8)), 1)
    tile = 6272
    nbuf = 4
    srows = 8

    x2 = x.reshape(B, C, HW)
    prop8 = proposal.reshape(B, srows, HW // srows)
    prop3 = proposal.reshape(B, 1, HW)
    w1t = (w1_0 * ln_g0[:, None]).T.astype(jnp.bfloat16)
    b1c = (b1_0 + ln_b0 @ w1_0)[:, None]
    w2t = w2_0.T.astype(jnp.bfloat16)
    b2c = b2_0[:, None]
    onesc = jnp.full((1, C), 1.0 / C, jnp.float32)

    out = pl.pallas_call(
        _make_kernel(B, C, HW, tile, kk, nbuf),
        in_specs=[
            pl.BlockSpec(memory_space=pltpu.MemorySpace.VMEM),
            pl.BlockSpec(memory_space=pltpu.MemorySpace.HBM),
            pl.BlockSpec(memory_space=pltpu.MemorySpace.HBM),
            pl.BlockSpec(memory_space=pltpu.MemorySpace.VMEM),
            pl.BlockSpec(memory_space=pltpu.MemorySpace.VMEM),
            pl.BlockSpec(memory_space=pltpu.MemorySpace.VMEM),
            pl.BlockSpec(memory_space=pltpu.MemorySpace.VMEM),
            pl.BlockSpec(memory_space=pltpu.MemorySpace.VMEM),
        ],
        out_specs=pl.BlockSpec(memory_space=pltpu.MemorySpace.HBM),
        out_shape=jax.ShapeDtypeStruct((B, C, HW), jnp.float32),
        scratch_shapes=[
            pltpu.VMEM((nbuf, C, tile), jnp.float32),
            pltpu.VMEM((nbuf, 1, tile), jnp.float32),
            pltpu.VMEM((nbuf, C, tile), jnp.float32),
            pltpu.SemaphoreType.DMA((nbuf,)),
            pltpu.SemaphoreType.DMA((nbuf,)),
            pltpu.SemaphoreType.DMA((nbuf,)),
            pltpu.SMEM((B, 2), jnp.int32),
        ],
    )(prop8, x2, prop3, w1t, b1c, w2t, b2c, onesc)
    return out.reshape(B, C, H, W)


# speedup vs baseline: 1.0080x; 1.0080x over previous
import math

import jax
import jax.numpy as jnp
from jax.experimental import pallas as pl
from jax.experimental.pallas import tpu as pltpu

_INT_MIN = -(2 ** 31)
_INT_MAX = 2 ** 31 - 1


def _sortable(f):
    b = jax.lax.bitcast_convert_type(f, jnp.int32)
    return jnp.where(b < 0,
                     jnp.bitwise_xor(jnp.bitwise_not(b), jnp.int32(_INT_MIN)),
                     b)


def _gelu(x):
    return x * jax.nn.sigmoid(
        x * (1.5957691216057308 + 0.07135481627272282 * (x * x)))


def _search_all(keys, kk, hw):
    nb, r, cc = keys.shape

    def cnt_gt(thrv):
        return jnp.sum((keys > thrv).astype(jnp.int32), axis=(1, 2),
                       keepdims=True)

    cnt_nonneg = jnp.sum((keys >= 0).astype(jnp.int32), axis=(1, 2),
                         keepdims=True)
    lo0 = jnp.where(cnt_nonneg >= kk,
                    jnp.zeros_like(cnt_nonneg),
                    jnp.full_like(cnt_nonneg, _INT_MIN))
    hi0 = jnp.where(cnt_nonneg >= kk,
                    jnp.full_like(cnt_nonneg, _INT_MAX),
                    jnp.full_like(cnt_nonneg, -1))

    def bs(i, lh):
        lo, hi = lh
        mid = lo + ((hi - lo) >> 1)
        c = cnt_gt(mid)
        take = c < kk
        return (jnp.where(take, lo, mid + 1), jnp.where(take, mid, hi))

    thr, _ = jax.lax.fori_loop(0, 31, bs, (lo0, hi0))
    rem = kk - cnt_gt(thr)
    eq = keys == thr

    ids = (jax.lax.broadcasted_iota(jnp.int32, (nb, r, cc), 1) * cc
           + jax.lax.broadcasted_iota(jnp.int32, (nb, r, cc), 2))

    def bs2(i, lh):
        lo2, hi2 = lh
        mid = (lo2 + hi2) >> 1
        c = jnp.sum((eq & (ids <= mid)).astype(jnp.int32), axis=(1, 2),
                    keepdims=True)
        take = c >= rem
        return (jnp.where(take, lo2, mid + 1), jnp.where(take, mid, hi2))

    z = jnp.zeros_like(thr)
    m, _ = jax.lax.fori_loop(0, 16, bs2, (z, z + (hw - 1)))
    return thr, jnp.where(rem > 0, m, z - 1)


def _make_kernel(nb, c, hw, tile, kk, nbuf):
    nt = hw // tile
    steps = nb * nt

    def body(prop8_ref, x_ref, p_ref, w1t_ref, b1_ref, w2t_ref, b2_ref,
             onesc_ref, out_ref, ibuf, pbuf, obuf, isem, psem, osem, sref):
        for k in range(nbuf):
            b0, t0 = k // nt, k % nt
            pltpu.make_async_copy(
                x_ref.at[b0, :, pl.ds(t0 * tile, tile)],
                ibuf.at[k], isem.at[k]).start()
            pltpu.make_async_copy(
                p_ref.at[b0, :, pl.ds(t0 * tile, tile)],
                pbuf.at[k], psem.at[k]).start()

        thr_all, m_all = _search_all(_sortable(prop8_ref[...]), kk, hw)
        biota = jax.lax.broadcasted_iota(jnp.int32, (nb, 1, 1), 0)
        for b in range(nb):
            sel = biota == b
            sref[b, 0] = jnp.sum(jnp.where(sel, thr_all, 0))
            sref[b, 1] = jnp.sum(jnp.where(sel, m_all, 0))

        def step(j, k):
            s = j * nbuf + k
            b = s // nt
            t = s % nt
            thr = sref[b, 0]
            m = sref[b, 1]

            pltpu.make_async_copy(
                x_ref.at[b, :, pl.ds(t * tile, tile)],
                ibuf.at[k], isem.at[k]).wait()
            pltpu.make_async_copy(
                p_ref.at[b, :, pl.ds(t * tile, tile)],
                pbuf.at[k], psem.at[k]).wait()

            keys_t = _sortable(pbuf[k])
            ids_t = (jax.lax.broadcasted_iota(jnp.int32, (1, tile), 1)
                     + t * tile)
            mask = ((keys_t > thr) | ((keys_t == thr) & (ids_t <= m))
                    ).astype(jnp.float32)

            h = ibuf[k]
            mu = jnp.mean(h, axis=0, keepdims=True)
            d = h - mu
            var = jnp.mean(d * d, axis=0, keepdims=True)
            zn = (d * jax.lax.rsqrt(var + 1e-5)).astype(jnp.bfloat16)
            z1 = jnp.dot(w1t_ref[...], zn,
                         preferred_element_type=jnp.float32) + b1_ref[...]
            a = _gelu(z1).astype(jnp.bfloat16)
            ff = jnp.dot(w2t_ref[...], a,
                         preferred_element_type=jnp.float32) + b2_ref[...]

            @pl.when(j > 0)
            def _wait_prev_out():
                pltpu.make_async_copy(
                    obuf.at[k], out_ref.at[b, :, pl.ds(t * tile, tile)],
                    osem.at[k]).wait()

            obuf[k] = h + mask * ff
            pltpu.make_async_copy(
                obuf.at[k], out_ref.at[b, :, pl.ds(t * tile, tile)],
                osem.at[k]).start()

            @pl.when(s + nbuf < steps)
            def _fetch_ahead():
                s2 = s + nbuf
                b2 = s2 // nt
                t2 = s2 % nt
                pltpu.make_async_copy(
                    x_ref.at[b2, :, pl.ds(t2 * tile, tile)],
                    ibuf.at[k], isem.at[k]).start()
                pltpu.make_async_copy(
                    p_ref.at[b2, :, pl.ds(t2 * tile, tile)],
                    pbuf.at[k], psem.at[k]).start()

        def loop_body(j, carry):
            for k in range(nbuf):
                step(j, k)
            return carry

        jax.lax.fori_loop(0, steps // nbuf, loop_body, 0)

        for k in range(nbuf):
            s = steps - nbuf + k
            b0 = s // nt
            t0 = s % nt
            pltpu.make_async_copy(
                obuf.at[k], out_ref.at[b0, :, pl.ds(t0 * tile, tile)],
                osem.at[k]).wait()

    return body


def kernel(x, proposal, ln_g0, ln_b0, w1_0, b1_0, w2_0, b2_0):
    B, C, H, W = x.shape
    HW = H * W
    HID = w1_0.shape[1]
    kk = max(int(math.ceil(HW * 0.8)), 1)
    tile = 6272
    nbuf = 4
    srows = 8

    x2 = x.reshape(B, C, HW)
    prop8 = proposal.reshape(B, srows, HW // srows)
    prop3 = proposal.reshape(B, 1, HW)
    w1t = (w1_0 * ln_g0[:, None]).T.astype(jnp.bfloat16)
    b1c = (b1_0 + ln_b0 @ w1_0)[:, None]
    w2t = w2_0.T.astype(jnp.bfloat16)
    b2c = b2_0[:, None]
    onesc = jnp.full((1, C), 1.0 / C, jnp.float32)

    out = pl.pallas_call(
        _make_kernel(B, C, HW, tile, kk, nbuf),
        in_specs=[
            pl.BlockSpec(memory_space=pltpu.MemorySpace.VMEM),
            pl.BlockSpec(memory_space=pltpu.MemorySpace.HBM),
            pl.BlockSpec(memory_space=pltpu.MemorySpace.HBM),
            pl.BlockSpec(memory_space=pltpu.MemorySpace.VMEM),
            pl.BlockSpec(memory_space=pltpu.MemorySpace.VMEM),
            pl.BlockSpec(memory_space=pltpu.MemorySpace.VMEM),
            pl.BlockSpec(memory_space=pltpu.MemorySpace.VMEM),
            pl.BlockSpec(memory_space=pltpu.MemorySpace.VMEM),
        ],
        out_specs=pl.BlockSpec(memory_space=pltpu.MemorySpace.HBM),
        out_shape=jax.ShapeDtypeStruct((B, C, HW), jnp.float32),
        scratch_shapes=[
            pltpu.VMEM((nbuf, C, tile), jnp.float32),
            pltpu.VMEM((nbuf, 1, tile), jnp.float32),
            pltpu.VMEM((nbuf, C, tile), jnp.float32),
            pltpu.SemaphoreType.DMA((nbuf,)),
            pltpu.SemaphoreType.DMA((nbuf,)),
            pltpu.SemaphoreType.DMA((nbuf,)),
            pltpu.SMEM((B, 2), jnp.int32),
        ],
    )(prop8, x2, prop3, w1t, b1c, w2t, b2c, onesc)
    return out.reshape(B, C, H, W)


# speedup vs baseline: 1.0378x; 1.0296x over previous
import math

import jax
import jax.numpy as jnp
from jax.experimental import pallas as pl
from jax.experimental.pallas import tpu as pltpu

_INT_MIN = -(2 ** 31)
_INT_MAX = 2 ** 31 - 1


def _sortable(f):
    b = jax.lax.bitcast_convert_type(f, jnp.int32)
    return jnp.where(b < 0,
                     jnp.bitwise_xor(jnp.bitwise_not(b), jnp.int32(_INT_MIN)),
                     b)


def _gelu(x):
    t = jnp.tanh(x * (0.7978845608028654 + 0.03567740813636141 * (x * x)))
    return 0.5 * x + (0.5 * x) * t


def _search_all(keys, kk, hw):
    nb, r, cc = keys.shape

    def cnt_gt(thrv):
        return jnp.sum((keys > thrv).astype(jnp.int32), axis=(1, 2),
                       keepdims=True)

    cnt_nonneg = jnp.sum((keys >= 0).astype(jnp.int32), axis=(1, 2),
                         keepdims=True)
    lo0 = jnp.where(cnt_nonneg >= kk,
                    jnp.zeros_like(cnt_nonneg),
                    jnp.full_like(cnt_nonneg, _INT_MIN))
    hi0 = jnp.where(cnt_nonneg >= kk,
                    jnp.full_like(cnt_nonneg, _INT_MAX),
                    jnp.full_like(cnt_nonneg, -1))

    def bs(i, lh):
        lo, hi = lh
        mid = lo + ((hi - lo) >> 1)
        c = cnt_gt(mid)
        take = c < kk
        return (jnp.where(take, lo, mid + 1), jnp.where(take, mid, hi))

    thr, _ = jax.lax.fori_loop(0, 31, bs, (lo0, hi0))
    rem = kk - cnt_gt(thr)
    eq = keys == thr

    ids = (jax.lax.broadcasted_iota(jnp.int32, (nb, r, cc), 1) * cc
           + jax.lax.broadcasted_iota(jnp.int32, (nb, r, cc), 2))

    def bs2(i, lh):
        lo2, hi2 = lh
        mid = (lo2 + hi2) >> 1
        c = jnp.sum((eq & (ids <= mid)).astype(jnp.int32), axis=(1, 2),
                    keepdims=True)
        take = c >= rem
        return (jnp.where(take, lo2, mid + 1), jnp.where(take, mid, hi2))

    z = jnp.zeros_like(thr)
    m, _ = jax.lax.fori_loop(0, 16, bs2, (z, z + (hw - 1)))
    return thr, jnp.where(rem > 0, m, z - 1)


def _make_kernel(nb, c, hw, tile, kk, nbuf):
    nt = hw // tile
    steps = nb * nt

    def body(prop8_ref, x_ref, p_ref, w1t_ref, b1_ref, w2t_ref, b2_ref,
             out_ref, ibuf, pbuf, obuf, isem, psem, osem, sref):
        for k in range(nbuf):
            b0, t0 = k // nt, k % nt
            pltpu.make_async_copy(
                x_ref.at[b0, :, pl.ds(t0 * tile, tile)],
                ibuf.at[k], isem.at[k]).start()
            pltpu.make_async_copy(
                p_ref.at[b0, :, pl.ds(t0 * tile, tile)],
                pbuf.at[k], psem.at[k]).start()

        thr_all, m_all = _search_all(_sortable(prop8_ref[...]), kk, hw)
        biota = jax.lax.broadcasted_iota(jnp.int32, (nb, 1, 1), 0)
        for b in range(nb):
            sel = biota == b
            sref[b, 0] = jnp.sum(jnp.where(sel, thr_all, 0))
            sref[b, 1] = jnp.sum(jnp.where(sel, m_all, 0))

        def step(j, k):
            s = j * nbuf + k
            b = s // nt
            t = s % nt
            thr = sref[b, 0]
            m = sref[b, 1]

            pltpu.make_async_copy(
                x_ref.at[b, :, pl.ds(t * tile, tile)],
                ibuf.at[k], isem.at[k]).wait()
            pltpu.make_async_copy(
                p_ref.at[b, :, pl.ds(t * tile, tile)],
                pbuf.at[k], psem.at[k]).wait()

            keys_t = _sortable(pbuf[k])
            ids_t = (jax.lax.broadcasted_iota(jnp.int32, (1, tile), 1)
                     + t * tile)
            mask = ((keys_t > thr) | ((keys_t == thr) & (ids_t <= m))
                    ).astype(jnp.float32)

            h = ibuf[k]
            mu = jnp.mean(h, axis=0, keepdims=True)
            d = h - mu
            var = jnp.mean(d * d, axis=0, keepdims=True)
            zn = d * jax.lax.rsqrt(var + 1e-5)
            z1 = jnp.dot(w1t_ref[...], zn,
                         preferred_element_type=jnp.float32) + b1_ref[...]
            a = _gelu(z1)
            ff = jnp.dot(w2t_ref[...], a,
                         preferred_element_type=jnp.float32) + b2_ref[...]

            @pl.when(j > 0)
            def _wait_prev_out():
                pltpu.make_async_copy(
                    obuf.at[k], out_ref.at[b, :, pl.ds(t * tile, tile)],
                    osem.at[k]).wait()

            obuf[k] = h + mask * ff
            pltpu.make_async_copy(
                obuf.at[k], out_ref.at[b, :, pl.ds(t * tile, tile)],
                osem.at[k]).start()

            @pl.when(s + nbuf < steps)
            def _fetch_ahead():
                s2 = s + nbuf
                b2 = s2 // nt
                t2 = s2 % nt
                pltpu.make_async_copy(
                    x_ref.at[b2, :, pl.ds(t2 * tile, tile)],
                    ibuf.at[k], isem.at[k]).start()
                pltpu.make_async_copy(
                    p_ref.at[b2, :, pl.ds(t2 * tile, tile)],
                    pbuf.at[k], psem.at[k]).start()

        def loop_body(j, carry):
            for k in range(nbuf):
                step(j, k)
            return carry

        jax.lax.fori_loop(0, steps // nbuf, loop_body, 0)

        for k in range(nbuf):
            s = steps - nbuf + k
            b0 = s // nt
            t0 = s % nt
            pltpu.make_async_copy(
                obuf.at[k], out_ref.at[b0, :, pl.ds(t0 * tile, tile)],
                osem.at[k]).wait()

    return body


def kernel(x, proposal, ln_g0, ln_b0, w1_0, b1_0, w2_0, b2_0):
    B, C, H, W = x.shape
    HW = H * W
    HID = w1_0.shape[1]
    kk = max(int(math.ceil(HW * 0.8)), 1)
    tile = 6272
    nbuf = 8
    srows = 8

    x2 = x.reshape(B, C, HW)
    prop8 = proposal.reshape(B, srows, HW // srows)
    prop3 = proposal.reshape(B, 1, HW)
    w1t = (w1_0 * ln_g0[:, None]).T
    b1c = (b1_0 + ln_b0 @ w1_0)[:, None]
    w2t = w2_0.T
    b2c = b2_0[:, None]

    out = pl.pallas_call(
        _make_kernel(B, C, HW, tile, kk, nbuf),
        in_specs=[
            pl.BlockSpec(memory_space=pltpu.MemorySpace.VMEM),
            pl.BlockSpec(memory_space=pltpu.MemorySpace.HBM),
            pl.BlockSpec(memory_space=pltpu.MemorySpace.HBM),
            pl.BlockSpec(memory_space=pltpu.MemorySpace.VMEM),
            pl.BlockSpec(memory_space=pltpu.MemorySpace.VMEM),
            pl.BlockSpec(memory_space=pltpu.MemorySpace.VMEM),
            pl.BlockSpec(memory_space=pltpu.MemorySpace.VMEM),
        ],
        out_specs=pl.BlockSpec(memory_space=pltpu.MemorySpace.HBM),
        out_shape=jax.ShapeDtypeStruct((B, C, HW), jnp.float32),
        scratch_shapes=[
            pltpu.VMEM((nbuf, C, tile), jnp.float32),
            pltpu.VMEM((nbuf, 1, tile), jnp.float32),
            pltpu.VMEM((nbuf, C, tile), jnp.float32),
            pltpu.SemaphoreType.DMA((nbuf,)),
            pltpu.SemaphoreType.DMA((nbuf,)),
            pltpu.SemaphoreType.DMA((nbuf,)),
            pltpu.SMEM((B, 2), jnp.int32),
        ],
    )(prop8, x2, prop3, w1t, b1c, w2t, b2c)
    return out.reshape(B, C, H, W)


# speedup vs baseline: 1.0409x; 1.0030x over previous
import math

import jax
import jax.numpy as jnp
from jax.experimental import pallas as pl
from jax.experimental.pallas import tpu as pltpu

_INT_MIN = -(2 ** 31)
_INT_MAX = 2 ** 31 - 1


def _sortable(f):
    b = jax.lax.bitcast_convert_type(f, jnp.int32)
    return jnp.where(b < 0,
                     jnp.bitwise_xor(jnp.bitwise_not(b), jnp.int32(_INT_MIN)),
                     b)


def _gelu(x):
    t = jnp.tanh(x * (0.7978845608028654 + 0.03567740813636141 * (x * x)))
    return 0.5 * x + (0.5 * x) * t


def _search_all(keys, kk, hw):
    nb, r, cc = keys.shape

    def cnt_gt(thrv):
        return jnp.sum((keys > thrv).astype(jnp.int32), axis=(1, 2),
                       keepdims=True)

    cnt_nonneg = jnp.sum((keys >= 0).astype(jnp.int32), axis=(1, 2),
                         keepdims=True)
    lo0 = jnp.where(cnt_nonneg >= kk,
                    jnp.zeros_like(cnt_nonneg),
                    jnp.full_like(cnt_nonneg, _INT_MIN))
    hi0 = jnp.where(cnt_nonneg >= kk,
                    jnp.full_like(cnt_nonneg, _INT_MAX),
                    jnp.full_like(cnt_nonneg, -1))

    def bs(i, lh):
        lo, hi = lh
        mid = lo + ((hi - lo) >> 1)
        c = cnt_gt(mid)
        take = c < kk
        return (jnp.where(take, lo, mid + 1), jnp.where(take, mid, hi))

    thr, _ = jax.lax.fori_loop(0, 31, bs, (lo0, hi0))
    rem = kk - cnt_gt(thr)
    eq = keys == thr

    ids = (jax.lax.broadcasted_iota(jnp.int32, (nb, r, cc), 1) * cc
           + jax.lax.broadcasted_iota(jnp.int32, (nb, r, cc), 2))

    def bs2(i, lh):
        lo2, hi2 = lh
        mid = (lo2 + hi2) >> 1
        c = jnp.sum((eq & (ids <= mid)).astype(jnp.int32), axis=(1, 2),
                    keepdims=True)
        take = c >= rem
        return (jnp.where(take, lo2, mid + 1), jnp.where(take, mid, hi2))

    z = jnp.zeros_like(thr)
    m, _ = jax.lax.fori_loop(0, 16, bs2, (z, z + (hw - 1)))
    return thr, jnp.where(rem > 0, m, z - 1)


def _make_kernel(nb, c, hw, tile, kk, nbuf):
    nt = hw // tile
    steps = nb * nt

    def body(prop8_ref, x_ref, p_ref, w1t_ref, b1_ref, w2t_ref, b2_ref,
             out_ref, ibuf, pbuf, obuf, isem, psem, osem, sref):
        for k in range(nbuf):
            b0, t0 = k // nt, k % nt
            pltpu.make_async_copy(
                x_ref.at[b0, :, pl.ds(t0 * tile, tile)],
                ibuf.at[k], isem.at[k]).start()
            pltpu.make_async_copy(
                p_ref.at[b0, :, pl.ds(t0 * tile, tile)],
                pbuf.at[k], psem.at[k]).start()

        thr_all, m_all = _search_all(_sortable(prop8_ref[...]), kk, hw)
        biota = jax.lax.broadcasted_iota(jnp.int32, (nb, 1, 1), 0)
        for b in range(nb):
            sel = biota == b
            sref[b, 0] = jnp.sum(jnp.where(sel, thr_all, 0))
            sref[b, 1] = jnp.sum(jnp.where(sel, m_all, 0))

        def step(j, k):
            s = j * nbuf + k
            b = s // nt
            t = s % nt
            thr = sref[b, 0]
            m = sref[b, 1]

            pltpu.make_async_copy(
                x_ref.at[b, :, pl.ds(t * tile, tile)],
                ibuf.at[k], isem.at[k]).wait()
            pltpu.make_async_copy(
                p_ref.at[b, :, pl.ds(t * tile, tile)],
                pbuf.at[k], psem.at[k]).wait()

            keys_t = _sortable(pbuf[k])
            ids_t = (jax.lax.broadcasted_iota(jnp.int32, (1, tile), 1)
                     + t * tile)
            mask = ((keys_t > thr) | ((keys_t == thr) & (ids_t <= m))
                    ).astype(jnp.float32)

            h = ibuf[k]
            mu = jnp.mean(h, axis=0, keepdims=True)
            d = h - mu
            var = jnp.mean(d * d, axis=0, keepdims=True)
            zn = d * jax.lax.rsqrt(var + 1e-5)
            z1 = jnp.dot(w1t_ref[...], zn,
                         preferred_element_type=jnp.float32) + b1_ref[...]
            a = _gelu(z1)
            ff = jnp.dot(w2t_ref[...], a,
                         preferred_element_type=jnp.float32) + b2_ref[...]

            @pl.when(j > 0)
            def _wait_prev_out():
                pltpu.make_async_copy(
                    obuf.at[k], out_ref.at[b, :, pl.ds(t * tile, tile)],
                    osem.at[k]).wait()

            obuf[k] = h + mask * ff
            pltpu.make_async_copy(
                obuf.at[k], out_ref.at[b, :, pl.ds(t * tile, tile)],
                osem.at[k]).start()

            @pl.when(s + nbuf < steps)
            def _fetch_ahead():
                s2 = s + nbuf
                b2 = s2 // nt
                t2 = s2 % nt
                pltpu.make_async_copy(
                    x_ref.at[b2, :, pl.ds(t2 * tile, tile)],
                    ibuf.at[k], isem.at[k]).start()
                pltpu.make_async_copy(
                    p_ref.at[b2, :, pl.ds(t2 * tile, tile)],
                    pbuf.at[k], psem.at[k]).start()

        def loop_body(j, carry):
            for k in range(nbuf):
                step(j, k)
            return carry

        jax.lax.fori_loop(0, steps // nbuf, loop_body, 0)

        for k in range(nbuf):
            s = steps - nbuf + k
            b0 = s // nt
            t0 = s % nt
            pltpu.make_async_copy(
                obuf.at[k], out_ref.at[b0, :, pl.ds(t0 * tile, tile)],
                osem.at[k]).wait()

    return body


def kernel(x, proposal, ln_g0, ln_b0, w1_0, b1_0, w2_0, b2_0):
    B, C, H, W = x.shape
    HW = H * W
    HID = w1_0.shape[1]
    kk = max(int(math.ceil(HW * 0.8)), 1)
    tile = 6272
    nbuf = 4
    srows = 8

    x2 = x.reshape(B, C, HW)
    prop8 = proposal.reshape(B, srows, HW // srows)
    prop3 = proposal.reshape(B, 1, HW)
    w1t = (w1_0 * ln_g0[:, None]).T
    b1c = (b1_0 + ln_b0 @ w1_0)[:, None]
    w2t = w2_0.T
    b2c = b2_0[:, None]

    out = pl.pallas_call(
        _make_kernel(B, C, HW, tile, kk, nbuf),
        in_specs=[
            pl.BlockSpec(memory_space=pltpu.MemorySpace.VMEM),
            pl.BlockSpec(memory_space=pltpu.MemorySpace.HBM),
            pl.BlockSpec(memory_space=pltpu.MemorySpace.HBM),
            pl.BlockSpec(memory_space=pltpu.MemorySpace.VMEM),
            pl.BlockSpec(memory_space=pltpu.MemorySpace.VMEM),
            pl.BlockSpec(memory_space=pltpu.MemorySpace.VMEM),
            pl.BlockSpec(memory_space=pltpu.MemorySpace.VMEM),
        ],
        out_specs=pl.BlockSpec(memory_space=pltpu.MemorySpace.HBM),
        out_shape=jax.ShapeDtypeStruct((B, C, HW), jnp.float32),
        scratch_shapes=[
            pltpu.VMEM((nbuf, C, tile), jnp.float32),
            pltpu.VMEM((nbuf, 1, tile), jnp.float32),
            pltpu.VMEM((nbuf, C, tile), jnp.float32),
            pltpu.SemaphoreType.DMA((nbuf,)),
            pltpu.SemaphoreType.DMA((nbuf,)),
            pltpu.SemaphoreType.DMA((nbuf,)),
            pltpu.SMEM((B, 2), jnp.int32),
        ],
    )(prop8, x2, prop3, w1t, b1c, w2t, b2c)
    return out.reshape(B, C, H, W)
